# baseline (speedup 1.0000x reference)
import functools

import jax
import jax.numpy as jnp
from jax.experimental import pallas as pl
from jax.experimental.pallas import tpu as pltpu

_LOG2E = 1.4426950408889634


def _dot(a, b, trans_b=False):
    dims = (((1,), (1 if trans_b else 0,)), ((), ()))
    return jax.lax.dot_general(a, b, dims, preferred_element_type=jnp.float32)


def _qkv_body(x_ref, xin_ref, w_ref, kv_ref, qhm_ref, kvhm_ref, xlhm_ref,
              *, c, nh, d, qscale):
    y = _dot(x_ref[...], w_ref[...])
    q = y[:, :c]
    k = y[:, c:2 * c]
    v = y[:, 2 * c:]
    qn = q / jnp.maximum(
        jnp.sqrt(jnp.sum(q * q, axis=-1, keepdims=True)), 1e-12)
    kn = k / jnp.maximum(
        jnp.sqrt(jnp.sum(k * k, axis=-1, keepdims=True)), 1e-12)
    kv_ref[:, 0, :] = kn
    kv_ref[:, 1, :] = v
    qs = qn * qscale
    for h in range(nh):
        sl = slice(h * d, (h + 1) * d)
        qhm_ref[h] = qs[:, sl].astype(jnp.bfloat16)
        kvhm_ref[h, 0] = kn[:, sl].astype(jnp.bfloat16)
        kvhm_ref[h, 1] = v[:, sl].astype(jnp.bfloat16)
        xlhm_ref[h, 0] = xin_ref[:, 0, sl].astype(jnp.bfloat16)
        xlhm_ref[h, 1] = xin_ref[:, 1, sl].astype(jnp.bfloat16)


def _attn_body(q_ref, kxl_ref, vxl_ref, kcu_ref, vcu_ref, relx_ref, relc_ref,
               wp_ref, bp_ref, o_ref, acc_ref, *, bt, nh):
    t = pl.program_id(1)
    h = pl.program_id(2)
    qh = q_ref[0]
    s1 = _dot(qh, kxl_ref[0, 0], trans_b=True) + relx_ref[...]
    p1 = jnp.exp2(s1)
    l = jnp.sum(p1, axis=-1, keepdims=True)
    oh = _dot(p1.astype(jnp.bfloat16), vxl_ref[0, 0])

    def _tile(j, oh, l, masked):
        ks = kcu_ref[0, 0, pl.ds(j * bt, bt), :]
        vs = vcu_ref[0, 0, pl.ds(j * bt, bt), :]
        s = _dot(qh, ks, trans_b=True) + relc_ref[j]
        if masked:
            rows = jax.lax.broadcasted_iota(jnp.int32, s.shape, 0)
            cols = jax.lax.broadcasted_iota(jnp.int32, s.shape, 1)
            s = jnp.where(cols <= rows, s, -1e30)
        p = jnp.exp2(s)
        l = l + jnp.sum(p, axis=-1, keepdims=True)
        oh = oh + _dot(p.astype(jnp.bfloat16), vs)
        return oh, l

    def _body(j, carry):
        return _tile(j, *carry, masked=False)

    oh, l = jax.lax.fori_loop(0, t, _body, (oh, l))
    oh, l = _tile(t, oh, l, masked=True)
    oh = oh / l
    contrib = _dot(oh, wp_ref[0])

    @pl.when(h == 0)
    def _():
        acc_ref[...] = contrib

    @pl.when(h > 0)
    def _():
        acc_ref[...] += contrib

    @pl.when(h == nh - 1)
    def _():
        o_ref[0] = acc_ref[...] + bp_ref[...]


def kernel(batch_file_idxs, relative_positions, x, xl_memory, Wq, Wk, Wv, Wp,
           bp, gate_bias):
    del batch_file_idxs, gate_bias
    B, T, C = x.shape
    TXL = xl_memory.shape[1]
    J = T + TXL
    H = 16
    D = C // H
    scale = float(D) ** -0.5

    assert T == TXL
    BT1 = min(512, B * T)
    nrows = B * T
    x2 = x.reshape(nrows, C)
    xl2 = xl_memory.reshape(nrows, 2, C)
    w_qkv = jnp.concatenate([Wq, Wk, Wv], axis=0).T
    kv_cur, q_hm, kv_hm, xl_hm = pl.pallas_call(
        functools.partial(_qkv_body, c=C, nh=H, d=D,
                          qscale=float(scale * _LOG2E)),
        grid=(nrows // BT1,),
        in_specs=[
            pl.BlockSpec((BT1, C), lambda i: (i, 0)),
            pl.BlockSpec((BT1, 2, C), lambda i: (i, 0, 0)),
            pl.BlockSpec((C, 3 * C), lambda i: (0, 0)),
        ],
        out_specs=[
            pl.BlockSpec((BT1, 2, C), lambda i: (i, 0, 0)),
            pl.BlockSpec((H, BT1, D), lambda i: (0, i, 0)),
            pl.BlockSpec((H, 2, BT1, D), lambda i: (0, 0, i, 0)),
            pl.BlockSpec((H, 2, BT1, D), lambda i: (0, 0, i, 0)),
        ],
        out_shape=[
            jax.ShapeDtypeStruct((nrows, 2, C), jnp.float32),
            jax.ShapeDtypeStruct((H, nrows, D), jnp.bfloat16),
            jax.ShapeDtypeStruct((H, 2, nrows, D), jnp.bfloat16),
            jax.ShapeDtypeStruct((H, 2, nrows, D), jnp.bfloat16),
        ],
    )(x2, xl2, w_qkv)

    rel = relative_positions.reshape(relative_positions.shape[-2],
                                     relative_positions.shape[-1])[-T:, -J:]
    rel = rel * jnp.float32(scale * _LOG2E)
    BT2 = min(256, T)
    nt = T // BT2
    rel_cur = rel[:, TXL:].reshape(T, nt, BT2).transpose(1, 0, 2)
    wp3 = Wp.T.reshape(H, D, C)
    bp2 = bp.reshape(1, C)

    out = pl.pallas_call(
        functools.partial(_attn_body, bt=BT2, nh=H),
        grid=(B, nt, H),
        in_specs=[
            pl.BlockSpec((1, BT2, D), lambda b, t, h: (h, b * nt + t, 0)),
            pl.BlockSpec((1, 1, TXL, D), lambda b, t, h: (h, 0, b, 0)),
            pl.BlockSpec((1, 1, TXL, D), lambda b, t, h: (h, 1, b, 0)),
            pl.BlockSpec((1, 1, T, D), lambda b, t, h: (h, 0, b, 0)),
            pl.BlockSpec((1, 1, T, D), lambda b, t, h: (h, 1, b, 0)),
            pl.BlockSpec((BT2, TXL), lambda b, t, h: (t, 0)),
            pl.BlockSpec((nt, BT2, BT2), lambda b, t, h: (0, t, 0)),
            pl.BlockSpec((1, D, C), lambda b, t, h: (h, 0, 0)),
            pl.BlockSpec((1, C), lambda b, t, h: (0, 0)),
        ],
        out_specs=pl.BlockSpec((1, BT2, C), lambda b, t, h: (b, t, 0)),
        out_shape=jax.ShapeDtypeStruct((B, T, C), jnp.float32),
        scratch_shapes=[pltpu.VMEM((BT2, C), jnp.float32)],
    )(q_hm, xl_hm, xl_hm, kv_hm, kv_hm, rel, rel_cur, wp3, bp2)

    return (out, kv_cur.reshape(B, T, 2, C))


# speedup vs baseline: 1.0235x; 1.0235x over previous
import functools

import jax
import jax.numpy as jnp
from jax.experimental import pallas as pl
from jax.experimental.pallas import tpu as pltpu

_LOG2E = 1.4426950408889634


def _dot(a, b, trans_b=False):
    dims = (((1,), (1 if trans_b else 0,)), ((), ()))
    return jax.lax.dot_general(a, b, dims, preferred_element_type=jnp.float32)


def _qkv_body(x_ref, xin_ref, w_ref, kv_ref, qhm_ref, kvhm_ref, xlhm_ref,
              *, c, nh, d, qscale):
    y = _dot(x_ref[...], w_ref[...])
    q = y[:, :c]
    k = y[:, c:2 * c]
    v = y[:, 2 * c:]
    qn = q / jnp.maximum(
        jnp.sqrt(jnp.sum(q * q, axis=-1, keepdims=True)), 1e-12)
    kn = k / jnp.maximum(
        jnp.sqrt(jnp.sum(k * k, axis=-1, keepdims=True)), 1e-12)
    kv_ref[:, 0, :] = kn
    kv_ref[:, 1, :] = v
    qs = qn * qscale
    for h in range(nh):
        sl = slice(h * d, (h + 1) * d)
        qhm_ref[h] = qs[:, sl].astype(jnp.bfloat16)
        kvhm_ref[h, 0] = kn[:, sl].astype(jnp.bfloat16)
        kvhm_ref[h, 1] = v[:, sl].astype(jnp.bfloat16)
        xlhm_ref[h, 0] = xin_ref[:, 0, sl].astype(jnp.bfloat16)
        xlhm_ref[h, 1] = xin_ref[:, 1, sl].astype(jnp.bfloat16)


def _attn_body(q_ref, kxl_ref, vxl_ref, kcu_ref, vcu_ref, rel_ref,
               wp_ref, bp_ref, o_ref, acc_ref, oh_ref, l_ref,
               *, bt, txl, nh, nt):
    t = pl.program_id(1)
    h = pl.program_id(2)
    qh = q_ref[0]
    s1 = _dot(qh, kxl_ref[0, 0], trans_b=True) + rel_ref[:, :txl]
    p1 = jnp.exp2(s1)
    l_ref[...] = jnp.sum(p1, axis=-1, keepdims=True)
    oh_ref[...] = _dot(p1.astype(jnp.bfloat16), vxl_ref[0, 0])

    for j in range(nt):
        @pl.when(j <= t)
        def _(j=j):
            ks = kcu_ref[0, 0, j * bt:(j + 1) * bt, :]
            vs = vcu_ref[0, 0, j * bt:(j + 1) * bt, :]
            s = _dot(qh, ks, trans_b=True) \
                + rel_ref[:, txl + j * bt:txl + (j + 1) * bt]
            rows = t * bt + jax.lax.broadcasted_iota(jnp.int32, s.shape, 0)
            cols = j * bt + jax.lax.broadcasted_iota(jnp.int32, s.shape, 1)
            s = jnp.where(cols <= rows, s, -1e30)
            p = jnp.exp2(s)
            l_ref[...] += jnp.sum(p, axis=-1, keepdims=True)
            oh_ref[...] += _dot(p.astype(jnp.bfloat16), vs)

    oh = oh_ref[...] / l_ref[...]
    contrib = _dot(oh, wp_ref[0])

    @pl.when(h == 0)
    def _():
        acc_ref[...] = contrib

    @pl.when(h > 0)
    def _():
        acc_ref[...] += contrib

    @pl.when(h == nh - 1)
    def _():
        o_ref[0] = acc_ref[...] + bp_ref[...]


def kernel(batch_file_idxs, relative_positions, x, xl_memory, Wq, Wk, Wv, Wp,
           bp, gate_bias):
    del batch_file_idxs, gate_bias
    B, T, C = x.shape
    TXL = xl_memory.shape[1]
    J = T + TXL
    H = 16
    D = C // H
    scale = float(D) ** -0.5

    assert T == TXL
    BT1 = min(512, B * T)
    nrows = B * T
    x2 = x.reshape(nrows, C)
    xl2 = xl_memory.reshape(nrows, 2, C)
    w_qkv = jnp.concatenate([Wq, Wk, Wv], axis=0).T
    kv_cur, q_hm, kv_hm, xl_hm = pl.pallas_call(
        functools.partial(_qkv_body, c=C, nh=H, d=D,
                          qscale=float(scale * _LOG2E)),
        grid=(nrows // BT1,),
        in_specs=[
            pl.BlockSpec((BT1, C), lambda i: (i, 0)),
            pl.BlockSpec((BT1, 2, C), lambda i: (i, 0, 0)),
            pl.BlockSpec((C, 3 * C), lambda i: (0, 0)),
        ],
        out_specs=[
            pl.BlockSpec((BT1, 2, C), lambda i: (i, 0, 0)),
            pl.BlockSpec((H, BT1, D), lambda i: (0, i, 0)),
            pl.BlockSpec((H, 2, BT1, D), lambda i: (0, 0, i, 0)),
            pl.BlockSpec((H, 2, BT1, D), lambda i: (0, 0, i, 0)),
        ],
        out_shape=[
            jax.ShapeDtypeStruct((nrows, 2, C), jnp.float32),
            jax.ShapeDtypeStruct((H, nrows, D), jnp.bfloat16),
            jax.ShapeDtypeStruct((H, 2, nrows, D), jnp.bfloat16),
            jax.ShapeDtypeStruct((H, 2, nrows, D), jnp.bfloat16),
        ],
    )(x2, xl2, w_qkv)

    rel = relative_positions.reshape(relative_positions.shape[-2],
                                     relative_positions.shape[-1])[-T:, -J:]
    rel = rel * jnp.float32(scale * _LOG2E)
    BT2 = min(256, T)
    nt = T // BT2
    wp3 = Wp.T.reshape(H, D, C)
    bp2 = bp.reshape(1, C)

    out = pl.pallas_call(
        functools.partial(_attn_body, bt=BT2, txl=TXL, nh=H, nt=nt),
        grid=(B, nt, H),
        in_specs=[
            pl.BlockSpec((1, BT2, D), lambda b, t, h: (h, b * nt + t, 0)),
            pl.BlockSpec((1, 1, TXL, D), lambda b, t, h: (h, 0, b, 0)),
            pl.BlockSpec((1, 1, TXL, D), lambda b, t, h: (h, 1, b, 0)),
            pl.BlockSpec((1, 1, T, D), lambda b, t, h: (h, 0, b, 0)),
            pl.BlockSpec((1, 1, T, D), lambda b, t, h: (h, 1, b, 0)),
            pl.BlockSpec((BT2, J), lambda b, t, h: (t, 0)),
            pl.BlockSpec((1, D, C), lambda b, t, h: (h, 0, 0)),
            pl.BlockSpec((1, C), lambda b, t, h: (0, 0)),
        ],
        out_specs=pl.BlockSpec((1, BT2, C), lambda b, t, h: (b, t, 0)),
        out_shape=jax.ShapeDtypeStruct((B, T, C), jnp.float32),
        scratch_shapes=[pltpu.VMEM((BT2, C), jnp.float32),
                        pltpu.VMEM((BT2, D), jnp.float32),
                        pltpu.VMEM((BT2, 1), jnp.float32)],
    )(q_hm, xl_hm, xl_hm, kv_hm, kv_hm, rel, wp3, bp2)

    return (out, kv_cur.reshape(B, T, 2, C))


# speedup vs baseline: 1.2516x; 1.2229x over previous
import functools

import jax
import jax.numpy as jnp
from jax.experimental import pallas as pl
from jax.experimental.pallas import tpu as pltpu

_LOG2E = 1.4426950408889634


def _dot(a, b, trans_b=False):
    dims = (((1,), (1 if trans_b else 0,)), ((), ()))
    return jax.lax.dot_general(a, b, dims, preferred_element_type=jnp.float32)


def _qkv_body(x_ref, xin_ref, w_ref, kv_ref, qhm_ref, kvhm_ref, xlhm_ref,
              *, c, nh, d, qscale):
    y = _dot(x_ref[...], w_ref[...])
    q = y[:, :c]
    k = y[:, c:2 * c]
    v = y[:, 2 * c:]
    qn = q / jnp.maximum(
        jnp.sqrt(jnp.sum(q * q, axis=-1, keepdims=True)), 1e-12)
    kn = k / jnp.maximum(
        jnp.sqrt(jnp.sum(k * k, axis=-1, keepdims=True)), 1e-12)
    kv_ref[:, 0, :] = kn
    kv_ref[:, 1, :] = v
    qs = qn * qscale
    for h in range(nh):
        sl = slice(h * d, (h + 1) * d)
        qhm_ref[h] = qs[:, sl].astype(jnp.bfloat16)
        kvhm_ref[h, 0] = kn[:, sl].astype(jnp.bfloat16)
        kvhm_ref[h, 1] = v[:, sl].astype(jnp.bfloat16)
        xlhm_ref[h, 0] = xin_ref[:, 0, sl].astype(jnp.bfloat16)
        xlhm_ref[h, 1] = xin_ref[:, 1, sl].astype(jnp.bfloat16)


def _attn_body(q_ref, kxl_ref, vxl_ref, kcu_ref, vcu_ref, rel_ref,
               wp_ref, bp_ref, o_ref, acc_ref, *, bt, txl, nh):
    t = pl.program_id(1)
    h = pl.program_id(2)
    qh = q_ref[0]
    s1 = _dot(qh, kxl_ref[0, 0], trans_b=True) + rel_ref[:, :txl]
    s2 = _dot(qh, kcu_ref[0, 0], trans_b=True) + rel_ref[:, txl:]
    rows = t * bt + jax.lax.broadcasted_iota(jnp.int32, s2.shape, 0)
    cols = jax.lax.broadcasted_iota(jnp.int32, s2.shape, 1)
    s2 = jnp.where(cols <= rows, s2, -1e30)
    p1 = jnp.exp2(s1)
    p2 = jnp.exp2(s2)
    l = (jnp.sum(p1, axis=-1, keepdims=True)
         + jnp.sum(p2, axis=-1, keepdims=True))
    oh = (_dot(p1.astype(jnp.bfloat16), vxl_ref[0, 0])
          + _dot(p2.astype(jnp.bfloat16), vcu_ref[0, 0])) / l
    contrib = _dot(oh, wp_ref[0])

    @pl.when(h == 0)
    def _():
        acc_ref[...] = contrib

    @pl.when(h > 0)
    def _():
        acc_ref[...] += contrib

    @pl.when(h == nh - 1)
    def _():
        o_ref[0] = acc_ref[...] + bp_ref[...]


def kernel(batch_file_idxs, relative_positions, x, xl_memory, Wq, Wk, Wv, Wp,
           bp, gate_bias):
    del batch_file_idxs, gate_bias
    B, T, C = x.shape
    TXL = xl_memory.shape[1]
    J = T + TXL
    H = 16
    D = C // H
    scale = float(D) ** -0.5

    assert T == TXL
    BT1 = min(512, B * T)
    nrows = B * T
    x2 = x.reshape(nrows, C)
    xl2 = xl_memory.reshape(nrows, 2, C)
    w_qkv = jnp.concatenate([Wq, Wk, Wv], axis=0).T
    kv_cur, q_hm, kv_hm, xl_hm = pl.pallas_call(
        functools.partial(_qkv_body, c=C, nh=H, d=D,
                          qscale=float(scale * _LOG2E)),
        grid=(nrows // BT1,),
        in_specs=[
            pl.BlockSpec((BT1, C), lambda i: (i, 0)),
            pl.BlockSpec((BT1, 2, C), lambda i: (i, 0, 0)),
            pl.BlockSpec((C, 3 * C), lambda i: (0, 0)),
        ],
        out_specs=[
            pl.BlockSpec((BT1, 2, C), lambda i: (i, 0, 0)),
            pl.BlockSpec((H, BT1, D), lambda i: (0, i, 0)),
            pl.BlockSpec((H, 2, BT1, D), lambda i: (0, 0, i, 0)),
            pl.BlockSpec((H, 2, BT1, D), lambda i: (0, 0, i, 0)),
        ],
        out_shape=[
            jax.ShapeDtypeStruct((nrows, 2, C), jnp.float32),
            jax.ShapeDtypeStruct((H, nrows, D), jnp.bfloat16),
            jax.ShapeDtypeStruct((H, 2, nrows, D), jnp.bfloat16),
            jax.ShapeDtypeStruct((H, 2, nrows, D), jnp.bfloat16),
        ],
    )(x2, xl2, w_qkv)

    rel = relative_positions.reshape(relative_positions.shape[-2],
                                     relative_positions.shape[-1])[-T:, -J:]
    rel = rel * jnp.float32(scale * _LOG2E)
    BT2 = min(512, T)
    nt = T // BT2
    wp3 = Wp.T.reshape(H, D, C)
    bp2 = bp.reshape(1, C)

    out = pl.pallas_call(
        functools.partial(_attn_body, bt=BT2, txl=TXL, nh=H),
        grid=(B, nt, H),
        in_specs=[
            pl.BlockSpec((1, BT2, D), lambda b, t, h: (h, b * nt + t, 0)),
            pl.BlockSpec((1, 1, TXL, D), lambda b, t, h: (h, 0, b, 0)),
            pl.BlockSpec((1, 1, TXL, D), lambda b, t, h: (h, 1, b, 0)),
            pl.BlockSpec((1, 1, T, D), lambda b, t, h: (h, 0, b, 0)),
            pl.BlockSpec((1, 1, T, D), lambda b, t, h: (h, 1, b, 0)),
            pl.BlockSpec((BT2, J), lambda b, t, h: (t, 0)),
            pl.BlockSpec((1, D, C), lambda b, t, h: (h, 0, 0)),
            pl.BlockSpec((1, C), lambda b, t, h: (0, 0)),
        ],
        out_specs=pl.BlockSpec((1, BT2, C), lambda b, t, h: (b, t, 0)),
        out_shape=jax.ShapeDtypeStruct((B, T, C), jnp.float32),
        scratch_shapes=[pltpu.VMEM((BT2, C), jnp.float32)],
    )(q_hm, xl_hm, xl_hm, kv_hm, kv_hm, rel, wp3, bp2)

    return (out, kv_cur.reshape(B, T, 2, C))


# speedup vs baseline: 1.3856x; 1.1071x over previous
import functools

import jax
import jax.numpy as jnp
from jax.experimental import pallas as pl
from jax.experimental.pallas import tpu as pltpu

_LOG2E = 1.4426950408889634


def _dot(a, b, trans_b=False):
    dims = (((1,), (1 if trans_b else 0,)), ((), ()))
    return jax.lax.dot_general(a, b, dims, preferred_element_type=jnp.float32)


def _qkv_body(x_ref, xin_ref, w_ref, kv_ref, qhm_ref, kvhm_ref, xlhm_ref,
              *, c, nh, d, qscale):
    y = _dot(x_ref[...], w_ref[...])
    q = y[:, :c]
    k = y[:, c:2 * c]
    v = y[:, 2 * c:]
    qn = q / jnp.maximum(
        jnp.sqrt(jnp.sum(q * q, axis=-1, keepdims=True)), 1e-12)
    kn = k / jnp.maximum(
        jnp.sqrt(jnp.sum(k * k, axis=-1, keepdims=True)), 1e-12)
    kv_ref[:, 0, :] = kn
    kv_ref[:, 1, :] = v
    qs = qn * qscale
    for h in range(nh):
        sl = slice(h * d, (h + 1) * d)
        qhm_ref[h] = qs[:, sl].astype(jnp.bfloat16)
        kvhm_ref[h, 0] = kn[:, sl].astype(jnp.bfloat16)
        kvhm_ref[h, 1] = v[:, sl].astype(jnp.bfloat16)
        xlhm_ref[h, 0] = xin_ref[:, 0, sl].astype(jnp.bfloat16)
        xlhm_ref[h, 1] = xin_ref[:, 1, sl].astype(jnp.bfloat16)


def _attn_body(q_ref, kxl_ref, vxl_ref, kcu_ref, vcu_ref, rel_ref,
               wp_ref, bp_ref, o_ref, acc_ref, *, bt, txl, nh):
    t = pl.program_id(1)
    h = pl.program_id(2)
    qh = q_ref[0]
    s1 = _dot(qh, kxl_ref[0, 0], trans_b=True) + rel_ref[:, :txl]
    s2 = _dot(qh, kcu_ref[0, 0], trans_b=True) + rel_ref[:, txl:]
    rows = t * bt + jax.lax.broadcasted_iota(jnp.int32, s2.shape, 0)
    cols = jax.lax.broadcasted_iota(jnp.int32, s2.shape, 1)
    s2 = jnp.where(cols <= rows, s2, -1e30)
    p1 = jnp.exp2(s1)
    p2 = jnp.exp2(s2)
    l = (jnp.sum(p1, axis=-1, keepdims=True)
         + jnp.sum(p2, axis=-1, keepdims=True))
    oh = (_dot(p1.astype(jnp.bfloat16), vxl_ref[0, 0])
          + _dot(p2.astype(jnp.bfloat16), vcu_ref[0, 0])) / l
    contrib = _dot(oh, wp_ref[0])

    @pl.when(h == 0)
    def _():
        acc_ref[...] = contrib

    @pl.when(h > 0)
    def _():
        acc_ref[...] += contrib

    @pl.when(h == nh - 1)
    def _():
        o_ref[0] = acc_ref[...] + bp_ref[...]


def kernel(batch_file_idxs, relative_positions, x, xl_memory, Wq, Wk, Wv, Wp,
           bp, gate_bias):
    del batch_file_idxs, gate_bias
    B, T, C = x.shape
    TXL = xl_memory.shape[1]
    J = T + TXL
    H = 16
    D = C // H
    scale = float(D) ** -0.5

    assert T == TXL
    BT1 = min(512, B * T)
    nrows = B * T
    x2 = x.reshape(nrows, C)
    xl2 = xl_memory.reshape(nrows, 2, C)
    w_qkv = jnp.concatenate([Wq, Wk, Wv], axis=0).T
    kv_cur, q_hm, kv_hm, xl_hm = pl.pallas_call(
        functools.partial(_qkv_body, c=C, nh=H, d=D,
                          qscale=float(scale * _LOG2E)),
        grid=(nrows // BT1,),
        in_specs=[
            pl.BlockSpec((BT1, C), lambda i: (i, 0)),
            pl.BlockSpec((BT1, 2, C), lambda i: (i, 0, 0)),
            pl.BlockSpec((C, 3 * C), lambda i: (0, 0)),
        ],
        out_specs=[
            pl.BlockSpec((BT1, 2, C), lambda i: (i, 0, 0)),
            pl.BlockSpec((H, BT1, D), lambda i: (0, i, 0)),
            pl.BlockSpec((H, 2, BT1, D), lambda i: (0, 0, i, 0)),
            pl.BlockSpec((H, 2, BT1, D), lambda i: (0, 0, i, 0)),
        ],
        out_shape=[
            jax.ShapeDtypeStruct((nrows, 2, C), jnp.float32),
            jax.ShapeDtypeStruct((H, nrows, D), jnp.bfloat16),
            jax.ShapeDtypeStruct((H, 2, nrows, D), jnp.bfloat16),
            jax.ShapeDtypeStruct((H, 2, nrows, D), jnp.bfloat16),
        ],
    )(x2, xl2, w_qkv)

    rel = relative_positions.reshape(relative_positions.shape[-2],
                                     relative_positions.shape[-1])[-T:, -J:]
    rel = rel * jnp.float32(scale * _LOG2E)
    BT2 = min(256, T)
    nt = T // BT2
    wp3 = Wp.T.reshape(H, D, C)
    bp2 = bp.reshape(1, C)

    out = pl.pallas_call(
        functools.partial(_attn_body, bt=BT2, txl=TXL, nh=H),
        grid=(B, nt, H),
        in_specs=[
            pl.BlockSpec((1, BT2, D), lambda b, t, h: (h, b * nt + t, 0)),
            pl.BlockSpec((1, 1, TXL, D), lambda b, t, h: (h, 0, b, 0)),
            pl.BlockSpec((1, 1, TXL, D), lambda b, t, h: (h, 1, b, 0)),
            pl.BlockSpec((1, 1, T, D), lambda b, t, h: (h, 0, b, 0)),
            pl.BlockSpec((1, 1, T, D), lambda b, t, h: (h, 1, b, 0)),
            pl.BlockSpec((BT2, J), lambda b, t, h: (t, 0)),
            pl.BlockSpec((1, D, C), lambda b, t, h: (h, 0, 0)),
            pl.BlockSpec((1, C), lambda b, t, h: (0, 0)),
        ],
        out_specs=pl.BlockSpec((1, BT2, C), lambda b, t, h: (b, t, 0)),
        out_shape=jax.ShapeDtypeStruct((B, T, C), jnp.float32),
        scratch_shapes=[pltpu.VMEM((BT2, C), jnp.float32)],
    )(q_hm, xl_hm, xl_hm, kv_hm, kv_hm, rel, wp3, bp2)

    return (out, kv_cur.reshape(B, T, 2, C))


# speedup vs baseline: 1.5049x; 1.0860x over previous
import functools

import jax
import jax.numpy as jnp
from jax.experimental import pallas as pl
from jax.experimental.pallas import tpu as pltpu

_LOG2E = 1.4426950408889634


def _dot(a, b, trans_b=False):
    dims = (((1,), (1 if trans_b else 0,)), ((), ()))
    return jax.lax.dot_general(a, b, dims, preferred_element_type=jnp.float32)


def _qkv_body(x_ref, xin_ref, w_ref, kv_ref, qhm_ref, kvhm_ref, xlhm_ref,
              *, c, nh, d, qscale):
    y = _dot(x_ref[...], w_ref[...])
    q = y[:, :c]
    k = y[:, c:2 * c]
    v = y[:, 2 * c:]
    qn = q / jnp.maximum(
        jnp.sqrt(jnp.sum(q * q, axis=-1, keepdims=True)), 1e-12)
    kn = k / jnp.maximum(
        jnp.sqrt(jnp.sum(k * k, axis=-1, keepdims=True)), 1e-12)
    kv_ref[:, 0, :] = kn
    kv_ref[:, 1, :] = v
    qs = qn * qscale
    for h in range(nh):
        sl = slice(h * d, (h + 1) * d)
        qhm_ref[h] = qs[:, sl].astype(jnp.bfloat16)
        kvhm_ref[h, 0] = kn[:, sl].astype(jnp.bfloat16)
        kvhm_ref[h, 1] = v[:, sl].astype(jnp.bfloat16)
        xlhm_ref[h, 0] = xin_ref[:, 0, sl].astype(jnp.bfloat16)
        xlhm_ref[h, 1] = xin_ref[:, 1, sl].astype(jnp.bfloat16)


def _attn_body(q_ref, kxl_ref, vxl_ref, kcu_ref, vcu_ref, relx_ref, relc_ref,
               wp_ref, bp_ref, o_ref, acc_ref, *, nh):
    h = pl.program_id(2)
    qh = q_ref[0]
    s1 = _dot(qh, kxl_ref[0, 0], trans_b=True) + relx_ref[...]
    s2 = _dot(qh, kcu_ref[0, 0], trans_b=True) + relc_ref[...]
    p1 = jnp.exp2(s1)
    p2 = jnp.exp2(s2)
    l = (jnp.sum(p1, axis=-1, keepdims=True)
         + jnp.sum(p2, axis=-1, keepdims=True))
    oh = (_dot(p1.astype(jnp.bfloat16), vxl_ref[0, 0])
          + _dot(p2.astype(jnp.bfloat16), vcu_ref[0, 0])) / l
    contrib = _dot(oh, wp_ref[0])

    @pl.when(h == 0)
    def _():
        acc_ref[...] = contrib

    @pl.when(h > 0)
    def _():
        acc_ref[...] += contrib

    @pl.when(h == nh - 1)
    def _():
        o_ref[0] = acc_ref[...] + bp_ref[...]


def kernel(batch_file_idxs, relative_positions, x, xl_memory, Wq, Wk, Wv, Wp,
           bp, gate_bias):
    del batch_file_idxs, gate_bias
    B, T, C = x.shape
    TXL = xl_memory.shape[1]
    J = T + TXL
    H = 16
    D = C // H
    scale = float(D) ** -0.5

    assert T == TXL
    BT1 = min(512, B * T)
    nrows = B * T
    x2 = x.reshape(nrows, C)
    xl2 = xl_memory.reshape(nrows, 2, C)
    w_qkv = jnp.concatenate([Wq, Wk, Wv], axis=0).T
    kv_cur, q_hm, kv_hm, xl_hm = pl.pallas_call(
        functools.partial(_qkv_body, c=C, nh=H, d=D,
                          qscale=float(scale * _LOG2E)),
        grid=(nrows // BT1,),
        in_specs=[
            pl.BlockSpec((BT1, C), lambda i: (i, 0)),
            pl.BlockSpec((BT1, 2, C), lambda i: (i, 0, 0)),
            pl.BlockSpec((C, 3 * C), lambda i: (0, 0)),
        ],
        out_specs=[
            pl.BlockSpec((BT1, 2, C), lambda i: (i, 0, 0)),
            pl.BlockSpec((H, BT1, D), lambda i: (0, i, 0)),
            pl.BlockSpec((H, 2, BT1, D), lambda i: (0, 0, i, 0)),
            pl.BlockSpec((H, 2, BT1, D), lambda i: (0, 0, i, 0)),
        ],
        out_shape=[
            jax.ShapeDtypeStruct((nrows, 2, C), jnp.float32),
            jax.ShapeDtypeStruct((H, nrows, D), jnp.bfloat16),
            jax.ShapeDtypeStruct((H, 2, nrows, D), jnp.bfloat16),
            jax.ShapeDtypeStruct((H, 2, nrows, D), jnp.bfloat16),
        ],
    )(x2, xl2, w_qkv)

    rel = relative_positions.reshape(relative_positions.shape[-2],
                                     relative_positions.shape[-1])[-T:, -J:]
    rel = rel * jnp.float32(scale * _LOG2E)
    rel = jnp.where(
        jnp.arange(J, dtype=jnp.int32)[None, :] - TXL
        > jnp.arange(T, dtype=jnp.int32)[:, None],
        jnp.float32(-1e30), rel)
    BT2 = min(256, T)
    nt = T // BT2
    wp3 = Wp.T.reshape(H, D, C)
    bp2 = bp.reshape(1, C)

    if nt == 8:
        groups = [(0, 2), (2, 2), (4, 4)]
    else:
        groups = [(0, nt)]
    outs = []
    for ts, ng in groups:
        L = min(T, (ts + ng) * BT2)
        o = pl.pallas_call(
            functools.partial(_attn_body, nh=H),
            grid=(B, ng, H),
            in_specs=[
                pl.BlockSpec((1, BT2, D),
                             lambda b, t, h, ts=ts: (h, b * nt + ts + t, 0)),
                pl.BlockSpec((1, 1, TXL, D), lambda b, t, h: (h, 0, b, 0)),
                pl.BlockSpec((1, 1, TXL, D), lambda b, t, h: (h, 1, b, 0)),
                pl.BlockSpec((1, 1, L, D),
                             lambda b, t, h, L=L: (h, 0, b * T // L, 0)),
                pl.BlockSpec((1, 1, L, D),
                             lambda b, t, h, L=L: (h, 1, b * T // L, 0)),
                pl.BlockSpec((BT2, TXL), lambda b, t, h, ts=ts: (ts + t, 0)),
                pl.BlockSpec((BT2, L),
                             lambda b, t, h, ts=ts, L=L: (ts + t, TXL // L)),
                pl.BlockSpec((1, D, C), lambda b, t, h: (h, 0, 0)),
                pl.BlockSpec((1, C), lambda b, t, h: (0, 0)),
            ],
            out_specs=pl.BlockSpec((1, BT2, C), lambda b, t, h: (b, t, 0)),
            out_shape=jax.ShapeDtypeStruct((B, ng * BT2, C), jnp.float32),
            scratch_shapes=[pltpu.VMEM((BT2, C), jnp.float32)],
        )(q_hm, xl_hm, xl_hm, kv_hm, kv_hm, rel, rel, wp3, bp2)
        outs.append(o)
    out = jnp.concatenate(outs, axis=1) if len(outs) > 1 else outs[0]

    return (out, kv_cur.reshape(B, T, 2, C))
